# baseline (speedup 1.0000x reference)
import functools

import jax
import jax.numpy as jnp
from jax import lax
from jax.experimental import pallas as pl
from jax.experimental.pallas import tpu as pltpu
from jax.experimental.pallas import tpu_sc as plsc

_NUM_EMB = 1000000
_D = 32
_R = 4
_SCALE = 10.0
_B = 4096 * 26
_NW = 32
_PER_W = _B // _NW
_C = 128
_NCH = _PER_W // _C


def _sc_body(x_hbm, emb_hbm, uflat_hbm, v_hbm, out_hbm,
             idx0, idx1, ix40, ix41, e0, e1, u0, u1, o0, o1, vs,
             sg0, sg1, sw0, sw1):
    wid = lax.axis_index("s") * 2 + lax.axis_index("c")
    base = wid * _PER_W

    slots = ((idx0, ix40, e0, u0, o0, sg0, sw0),
             (idx1, ix41, e1, u1, o1, sg1, sw1))

    pltpu.sync_copy(v_hbm, vs)
    for r in range(_R):
        for h in range(2):
            vs[r, pl.ds(16 * h, 16)] = vs[r, pl.ds(16 * h, 16)] * _SCALE

    def fire(c, s):
        idx_v, ix4_v, e_v, u_v, _, sg, _ = slots[s]
        pltpu.sync_copy(x_hbm.at[pl.ds(base + c * _C, _C)], idx_v)
        for q in range(_C // 16):
            iv4 = idx_v[pl.ds(16 * q, 16)] * 4
            for r in range(_R):
                ix4_v[r, pl.ds(16 * q, 16)] = iv4 + r
        pltpu.async_copy(emb_hbm.at[idx_v], e_v, sg)
        for r in range(_R):
            pltpu.async_copy(uflat_hbm.at[ix4_v.at[r]], u_v.at[r], sg)

    def wait_gathers(s):
        idx_v, ix4_v, e_v, u_v, _, sg, _ = slots[s]
        pltpu.make_async_copy(emb_hbm.at[idx_v], e_v, sg).wait()
        for r in range(_R):
            pltpu.make_async_copy(uflat_hbm.at[ix4_v.at[r]],
                                  u_v.at[r], sg).wait()

    def write_out(c, s):
        _, _, _, _, o_v, _, sw = slots[s]
        pltpu.async_copy(o_v, out_hbm.at[pl.ds(base + c * _C, _C)], sw)

    def wait_write(c, s):
        _, _, _, _, o_v, _, sw = slots[s]
        pltpu.make_async_copy(o_v, out_hbm.at[pl.ds(base + c * _C, _C)],
                              sw).wait()

    def compute(s):
        _, _, e_v, u_v, o_v, _, _ = slots[s]

        def row_group(g, _, e_v=e_v, u_v=u_v, o_v=o_v):
            ur = [u_v[r, pl.ds(16 * g, 16)] for r in range(_R)]
            for j in range(16):
                i = g * 16 + j
                for h in range(2):
                    sl = pl.ds(16 * h, 16)
                    acc = e_v[i, sl] * _SCALE
                    for r in range(_R):
                        acc = acc + ur[r][j] * vs[r, sl]
                    o_v[i, sl] = acc
            return 0

        lax.fori_loop(0, _C // 16, row_group, 0)

    fire(0, 0)
    fire(1, 1)

    def pair_body(p, _):
        for s in range(2):
            c = p * 2 + s
            wait_gathers(s)

            @pl.when(p >= 1)
            def _():
                wait_write(c - 2, s)

            compute(s)
            write_out(c, s)

            @pl.when(c + 2 < _NCH)
            def _():
                fire(c + 2, s)
        return 0

    lax.fori_loop(0, _NCH // 2, pair_body, 0)
    wait_write(_NCH - 2, 0)
    wait_write(_NCH - 1, 1)


@jax.jit
def kernel(x, emb_weight, lora_U, lora_V):
    x_flat = x.reshape(_B).astype(jnp.int32)
    u_flat = lora_U.reshape(_NUM_EMB * _R)
    sc_call = pl.kernel(
        _sc_body,
        out_type=jax.ShapeDtypeStruct((_B, _D), jnp.float32),
        mesh=plsc.VectorSubcoreMesh(core_axis_name="c", subcore_axis_name="s"),
        scratch_types=[
            pltpu.VMEM((_C,), jnp.int32),
            pltpu.VMEM((_C,), jnp.int32),
            pltpu.VMEM((_R, _C), jnp.int32),
            pltpu.VMEM((_R, _C), jnp.int32),
            pltpu.VMEM((_C, _D), jnp.float32),
            pltpu.VMEM((_C, _D), jnp.float32),
            pltpu.VMEM((_R, _C), jnp.float32),
            pltpu.VMEM((_R, _C), jnp.float32),
            pltpu.VMEM((_C, _D), jnp.float32),
            pltpu.VMEM((_C, _D), jnp.float32),
            pltpu.VMEM((_R, _D), jnp.float32),
            pltpu.SemaphoreType.DMA,
            pltpu.SemaphoreType.DMA,
            pltpu.SemaphoreType.DMA,
            pltpu.SemaphoreType.DMA,
        ],
        compiler_params=pltpu.CompilerParams(use_tc_tiling_on_sc=False,
                                             needs_layout_passes=False),
    )
    out = sc_call(x_flat, emb_weight, u_flat, lora_V)
    return out.reshape(x.shape[0], x.shape[1], _D)


# speedup vs baseline: 1.1363x; 1.1363x over previous
import functools

import jax
import jax.numpy as jnp
from jax import lax
from jax.experimental import pallas as pl
from jax.experimental.pallas import tpu as pltpu
from jax.experimental.pallas import tpu_sc as plsc

_NUM_EMB = 1000000
_D = 32
_R = 4
_SCALE = 10.0
_B = 4096 * 26
_NW = 32
_PER_W = _B // _NW
_C = 128
_NCH = _PER_W // _C
_BLK = _NUM_EMB // 32


def _sc_body(x_hbm, emb_hbm, u0_hbm, u1_hbm, u2_hbm, u3_hbm, v_hbm, out_hbm,
             idx0, idx1, ixe0, ixe1, e0, e1, u0, u1, o0, o1, vs,
             sg0, sg1, sw0, sw1):
    wid = lax.axis_index("s") * 2 + lax.axis_index("c")
    base = wid * _PER_W
    u_hbms = (u0_hbm, u1_hbm, u2_hbm, u3_hbm)

    slots = ((idx0, ixe0, e0, u0, o0, sg0, sw0),
             (idx1, ixe1, e1, u1, o1, sg1, sw1))

    pltpu.sync_copy(v_hbm, vs)
    for r in range(_R):
        for h in range(2):
            vs[r, pl.ds(16 * h, 16)] = vs[r, pl.ds(16 * h, 16)] * _SCALE

    def fire(c, s):
        idx_v, ixe_v, e_v, u_v, _, sg, _ = slots[s]
        pltpu.sync_copy(x_hbm.at[pl.ds(base + c * _C, _C)], idx_v)
        for q in range(_C // 16):
            iv = idx_v[pl.ds(16 * q, 16)]
            rho = (iv & 31) * _BLK + (iv >> 5)
            ixe_v[pl.ds(16 * q, 16)] = rho
        pltpu.async_copy(emb_hbm.at[ixe_v], e_v, sg)
        for r in range(_R):
            pltpu.async_copy(u_hbms[r].at[idx_v], u_v.at[r], sg)

    def wait_gathers(s):
        idx_v, ixe_v, e_v, u_v, _, sg, _ = slots[s]
        pltpu.make_async_copy(emb_hbm.at[ixe_v], e_v, sg).wait()
        for r in range(_R):
            pltpu.make_async_copy(u_hbms[r].at[idx_v], u_v.at[r], sg).wait()

    def write_out(c, s):
        _, _, _, _, o_v, _, sw = slots[s]
        pltpu.async_copy(o_v, out_hbm.at[pl.ds((base + c * _C) // 4, _C // 4)],
                         sw)

    def wait_write(c, s):
        _, _, _, _, o_v, _, sw = slots[s]
        pltpu.make_async_copy(
            o_v, out_hbm.at[pl.ds((base + c * _C) // 4, _C // 4)], sw).wait()

    def compute(s):
        _, _, e_v, u_v, o_v, _, _ = slots[s]

        def row_group(g, _, e_v=e_v, u_v=u_v, o_v=o_v):
            ur = [u_v[r, pl.ds(16 * g, 16)] for r in range(_R)]
            for j in range(16):
                i = g * 16 + j
                for h in range(2):
                    sl = pl.ds(16 * h, 16)
                    acc = e_v[i, sl] * _SCALE
                    for r in range(_R):
                        acc = acc + ur[r][j] * vs[r, sl]
                    o_v[4 * g + j // 4, pl.ds(32 * (j % 4) + 16 * h, 16)] = acc
            return 0

        lax.fori_loop(0, _C // 16, row_group, 0)

    fire(0, 0)
    fire(1, 1)

    def pair_body(p, _):
        for s in range(2):
            c = p * 2 + s
            wait_gathers(s)

            @pl.when(p >= 1)
            def _():
                wait_write(c - 2, s)

            compute(s)
            write_out(c, s)

            @pl.when(c + 2 < _NCH)
            def _():
                fire(c + 2, s)
        return 0

    lax.fori_loop(0, _NCH // 2, pair_body, 0)
    wait_write(_NCH - 2, 0)
    wait_write(_NCH - 1, 1)


@jax.jit
def kernel(x, emb_weight, lora_U, lora_V):
    x_flat = x.reshape(_B).astype(jnp.int32)
    emb_p = emb_weight.reshape(_BLK, 32, _D).transpose(1, 0, 2).reshape(
        _NUM_EMB, _D)
    u_t = lora_U.T
    u_cols = [u_t[r] for r in range(_R)]
    sc_call = pl.kernel(
        _sc_body,
        out_type=jax.ShapeDtypeStruct((_B // 4, 128), jnp.float32),
        mesh=plsc.VectorSubcoreMesh(core_axis_name="c", subcore_axis_name="s"),
        scratch_types=[
            pltpu.VMEM((_C,), jnp.int32),
            pltpu.VMEM((_C,), jnp.int32),
            pltpu.VMEM((_C,), jnp.int32),
            pltpu.VMEM((_C,), jnp.int32),
            pltpu.VMEM((_C, _D), jnp.float32),
            pltpu.VMEM((_C, _D), jnp.float32),
            pltpu.VMEM((_R, _C), jnp.float32),
            pltpu.VMEM((_R, _C), jnp.float32),
            pltpu.VMEM((_C // 4, 128), jnp.float32),
            pltpu.VMEM((_C // 4, 128), jnp.float32),
            pltpu.VMEM((_R, _D), jnp.float32),
            pltpu.SemaphoreType.DMA,
            pltpu.SemaphoreType.DMA,
            pltpu.SemaphoreType.DMA,
            pltpu.SemaphoreType.DMA,
        ],
        compiler_params=pltpu.CompilerParams(use_tc_tiling_on_sc=False,
                                             needs_layout_passes=False),
    )
    out = sc_call(x_flat, emb_p, *u_cols, lora_V)
    return out.reshape(x.shape[0], x.shape[1], _D)


# speedup vs baseline: 2.8328x; 2.4930x over previous
import functools

import jax
import jax.numpy as jnp
from jax import lax
from jax.experimental import pallas as pl
from jax.experimental.pallas import tpu as pltpu
from jax.experimental.pallas import tpu_sc as plsc

_NUM_EMB = 1000000
_D = 32
_R = 4
_SCALE = 10.0
_B = 4096 * 26
_NW = 32
_PER_W = _B // _NW
_C = 128
_NCH = _PER_W // _C


def _sc_body(x_hbm, emb_hbm, u0_hbm, u1_hbm, u2_hbm, u3_hbm, v_hbm, out_hbm,
             idx0, idx1, ixe0, ixe1, im0, im1, e0, e1,
             ua0, ub0, uc0, ud0, ua1, ub1, uc1, ud1, o0, o1, vs,
             sg0, sg1, sw0, sw1):
    wid = lax.axis_index("s") * 2 + lax.axis_index("c")
    base = wid * _PER_W
    u_hbms = (u0_hbm, u1_hbm, u2_hbm, u3_hbm)

    slots = ((idx0, ixe0, im0, e0, (ua0, ub0, uc0, ud0), o0, sg0, sw0),
             (idx1, ixe1, im1, e1, (ua1, ub1, uc1, ud1), o1, sg1, sw1))

    pltpu.sync_copy(v_hbm, vs)
    for r in range(_R):
        for h in range(2):
            vs[r, pl.ds(16 * h, 16)] = vs[r, pl.ds(16 * h, 16)] * _SCALE

    def fire(c, s):
        idx_v, ixe_v, im_v, e_v, u_v, _, sg, _ = slots[s]
        off = pl.multiple_of(base + c * _C, _C)
        pltpu.sync_copy(x_hbm.at[pl.ds(off, _C)], idx_v)
        for q in range(_C // 16):
            iv = idx_v[pl.ds(16 * q, 16)]
            ixe_v[pl.ds(16 * q, 16)] = iv >> 2
            im_v[pl.ds(16 * q, 16)] = iv & 3
        pltpu.async_copy(emb_hbm.at[ixe_v], e_v, sg)
        for r in range(_R):
            pltpu.async_copy(u_hbms[r].at[idx_v], u_v[r], sg)

    def wait_gathers(s):
        idx_v, ixe_v, _, e_v, u_v, _, sg, _ = slots[s]
        pltpu.make_async_copy(emb_hbm.at[ixe_v], e_v, sg).wait()
        for r in range(_R):
            pltpu.make_async_copy(u_hbms[r].at[idx_v], u_v[r], sg).wait()

    def write_out(c, s):
        o_v, sw = slots[s][5], slots[s][7]
        off = pl.multiple_of((base + c * _C) // 4, _C // 4)
        pltpu.async_copy(o_v, out_hbm.at[pl.ds(off, _C // 4)], sw)

    def wait_write(c, s):
        o_v, sw = slots[s][5], slots[s][7]
        off = pl.multiple_of((base + c * _C) // 4, _C // 4)
        pltpu.make_async_copy(
            o_v, out_hbm.at[pl.ds(off, _C // 4)], sw).wait()

    def compute(s):
        _, _, im_v, e_v, u_v, o_v, _, _ = slots[s]

        def row_group(g, _, im_v=im_v, e_v=e_v, u_v=u_v, o_v=o_v):
            ur = [u_v[r][pl.ds(16 * g, 16)] for r in range(_R)]
            imvec = im_v[pl.ds(16 * g, 16)]
            for j in range(16):
                i = g * 16 + j
                mj = imvec[j] * 32
                for h in range(2):
                    acc = e_v[i, pl.ds(mj + 16 * h, 16)] * _SCALE
                    for r in range(_R):
                        acc = acc + ur[r][j] * vs[r, pl.ds(16 * h, 16)]
                    o_v[4 * g + j // 4, pl.ds(32 * (j % 4) + 16 * h, 16)] = acc
            return 0

        lax.fori_loop(0, _C // 16, row_group, 0)

    fire(0, 0)
    fire(1, 1)

    def pair_body(p, _):
        for s in range(2):
            c = p * 2 + s
            wait_gathers(s)

            @pl.when(p >= 1)
            def _():
                wait_write(c - 2, s)

            compute(s)
            write_out(c, s)

            @pl.when(c + 2 < _NCH)
            def _():
                fire(c + 2, s)
        return 0

    lax.fori_loop(0, _NCH // 2, pair_body, 0)
    wait_write(_NCH - 2, 0)
    wait_write(_NCH - 1, 1)


@jax.jit
def kernel(x, emb_weight, lora_U, lora_V):
    x_flat = x.reshape(_B).astype(jnp.int32)
    emb128 = emb_weight.reshape(_NUM_EMB // 4, 128)
    u_t = lora_U.T
    u_cols = [u_t[r] for r in range(_R)]
    v_pad = jnp.zeros((_R, 128), jnp.float32).at[:, :_D].set(lora_V)
    sc_call = pl.kernel(
        _sc_body,
        out_type=jax.ShapeDtypeStruct((_B // 4, 128), jnp.float32),
        mesh=plsc.VectorSubcoreMesh(core_axis_name="c", subcore_axis_name="s"),
        scratch_types=[
            pltpu.VMEM((_C,), jnp.int32),
            pltpu.VMEM((_C,), jnp.int32),
            pltpu.VMEM((_C,), jnp.int32),
            pltpu.VMEM((_C,), jnp.int32),
            pltpu.VMEM((_C,), jnp.int32),
            pltpu.VMEM((_C,), jnp.int32),
            pltpu.VMEM((_C, 128), jnp.float32),
            pltpu.VMEM((_C, 128), jnp.float32),
            pltpu.VMEM((_C,), jnp.float32),
            pltpu.VMEM((_C,), jnp.float32),
            pltpu.VMEM((_C,), jnp.float32),
            pltpu.VMEM((_C,), jnp.float32),
            pltpu.VMEM((_C,), jnp.float32),
            pltpu.VMEM((_C,), jnp.float32),
            pltpu.VMEM((_C,), jnp.float32),
            pltpu.VMEM((_C,), jnp.float32),
            pltpu.VMEM((_C // 4, 128), jnp.float32),
            pltpu.VMEM((_C // 4, 128), jnp.float32),
            pltpu.VMEM((_R, 128), jnp.float32),
            pltpu.SemaphoreType.DMA,
            pltpu.SemaphoreType.DMA,
            pltpu.SemaphoreType.DMA,
            pltpu.SemaphoreType.DMA,
        ],
        compiler_params=pltpu.CompilerParams(use_tc_tiling_on_sc=True,
                                             needs_layout_passes=False),
    )
    out = sc_call(x_flat, emb128, *u_cols, v_pad)
    return out.reshape(x.shape[0], x.shape[1], _D)


# speedup vs baseline: 2.8344x; 1.0006x over previous
import functools

import jax
import jax.numpy as jnp
from jax import lax
from jax.experimental import pallas as pl
from jax.experimental.pallas import tpu as pltpu
from jax.experimental.pallas import tpu_sc as plsc

_NUM_EMB = 1000000
_D = 32
_R = 4
_SCALE = 10.0
_B = 4096 * 26
_NW = 32
_PER_W = _B // _NW
_C = 128
_NCH = _PER_W // _C


def _sc_body(x_hbm, emb_hbm, u0_hbm, u1_hbm, u2_hbm, u3_hbm, v_hbm, out_hbm,
             idx0, idx1, ixe0, ixe1, im0, im1, e0, e1,
             ua0, ub0, uc0, ud0, ua1, ub1, uc1, ud1, o0, o1, vs,
             sg0, sg1, sw0, sw1):
    wid = lax.axis_index("s") * 2 + lax.axis_index("c")
    base = wid * _PER_W
    u_hbms = (u0_hbm, u1_hbm, u2_hbm, u3_hbm)

    slots = ((idx0, ixe0, im0, e0, (ua0, ub0, uc0, ud0), o0, sg0, sw0),
             (idx1, ixe1, im1, e1, (ua1, ub1, uc1, ud1), o1, sg1, sw1))

    pltpu.sync_copy(v_hbm, vs)
    for r in range(_R):
        for h in range(2):
            vs[r, pl.ds(16 * h, 16)] = vs[r, pl.ds(16 * h, 16)] * _SCALE

    def fire(c, s):
        idx_v, ixe_v, im_v, e_v, u_v, _, sg, _ = slots[s]
        off = pl.multiple_of(base + c * _C, _C)
        pltpu.sync_copy(x_hbm.at[pl.ds(off, _C)], idx_v)
        for q in range(_C // 16):
            iv = idx_v[pl.ds(16 * q, 16)]
            ixe_v[pl.ds(16 * q, 16)] = iv >> 2
            im_v[pl.ds(16 * q, 16)] = iv & 3
        pltpu.async_copy(emb_hbm.at[ixe_v], e_v, sg)
        for r in range(_R):
            pltpu.async_copy(u_hbms[r].at[idx_v], u_v[r], sg)

    def wait_gathers(s):
        idx_v, ixe_v, _, e_v, u_v, _, sg, _ = slots[s]
        pltpu.make_async_copy(emb_hbm.at[ixe_v], e_v, sg).wait()
        for r in range(_R):
            pltpu.make_async_copy(u_hbms[r].at[idx_v], u_v[r], sg).wait()

    def write_out(c, s):
        o_v, sw = slots[s][5], slots[s][7]
        off = pl.multiple_of((base + c * _C) // 4, _C // 4)
        pltpu.async_copy(o_v, out_hbm.at[pl.ds(off, _C // 4)], sw)

    def wait_write(c, s):
        o_v, sw = slots[s][5], slots[s][7]
        off = pl.multiple_of((base + c * _C) // 4, _C // 4)
        pltpu.make_async_copy(
            o_v, out_hbm.at[pl.ds(off, _C // 4)], sw).wait()

    def compute(s):
        _, _, im_v, e_v, u_v, o_v, _, _ = slots[s]

        def row_group(g, _, im_v=im_v, e_v=e_v, u_v=u_v, o_v=o_v):
            ur = [u_v[r][pl.ds(16 * g, 16)] for r in range(_R)]
            imvec = im_v[pl.ds(16 * g, 16)]
            for j in range(16):
                i = g * 16 + j
                mj = imvec[j] * 32
                for h in range(2):
                    acc = e_v[i, pl.ds(mj + 16 * h, 16)] * _SCALE
                    for r in range(_R):
                        acc = acc + ur[r][j] * vs[r, pl.ds(16 * h, 16)]
                    o_v[4 * g + j // 4, pl.ds(32 * (j % 4) + 16 * h, 16)] = acc
            return 0

        lax.fori_loop(0, _C // 16, row_group, 0)

    fire(0, 0)
    fire(1, 1)

    def pair_body(p, _):
        for s in range(2):
            c = p * 2 + s
            wait_gathers(s)

            @pl.when(p >= 1)
            def _():
                wait_write(c - 2, s)

            compute(s)
            write_out(c, s)

            @pl.when(c + 2 < _NCH)
            def _():
                fire(c + 2, s)
        return 0

    lax.fori_loop(0, _NCH // 2, pair_body, 0)
    wait_write(_NCH - 2, 0)
    wait_write(_NCH - 1, 1)


@jax.jit
def kernel(x, emb_weight, lora_U, lora_V):
    x_flat = x.reshape(_B).astype(jnp.int32)
    emb128 = emb_weight.reshape(_NUM_EMB // 4, 128)
    u_t = lora_U.T
    u_cols = [u_t[r] for r in range(_R)]
    v_pad = jnp.zeros((_R, 128), jnp.float32).at[:, :_D].set(lora_V)
    sc_call = pl.kernel(
        _sc_body,
        out_type=jax.ShapeDtypeStruct((_B // 4, 128), jnp.float32),
        mesh=plsc.VectorSubcoreMesh(core_axis_name="c", subcore_axis_name="s"),
        scratch_types=[
            pltpu.VMEM((_C,), jnp.int32),
            pltpu.VMEM((_C,), jnp.int32),
            pltpu.VMEM((_C,), jnp.int32),
            pltpu.VMEM((_C,), jnp.int32),
            pltpu.VMEM((_C,), jnp.int32),
            pltpu.VMEM((_C,), jnp.int32),
            pltpu.VMEM((_C, 128), jnp.float32),
            pltpu.VMEM((_C, 128), jnp.float32),
            pltpu.VMEM((_C,), jnp.float32),
            pltpu.VMEM((_C,), jnp.float32),
            pltpu.VMEM((_C,), jnp.float32),
            pltpu.VMEM((_C,), jnp.float32),
            pltpu.VMEM((_C,), jnp.float32),
            pltpu.VMEM((_C,), jnp.float32),
            pltpu.VMEM((_C,), jnp.float32),
            pltpu.VMEM((_C,), jnp.float32),
            pltpu.VMEM((_C // 4, 128), jnp.float32),
            pltpu.VMEM((_C // 4, 128), jnp.float32),
            pltpu.VMEM((_R, 128), jnp.float32),
            pltpu.SemaphoreType.DMA,
            pltpu.SemaphoreType.DMA,
            pltpu.SemaphoreType.DMA,
            pltpu.SemaphoreType.DMA,
        ],
        compiler_params=pltpu.CompilerParams(use_tc_tiling_on_sc=False,
                                             needs_layout_passes=False),
    )
    out = sc_call(x_flat, emb128, *u_cols, v_pad)
    return out.reshape(x.shape[0], x.shape[1], _D)


# speedup vs baseline: 2.8362x; 1.0006x over previous
import jax
import jax.numpy as jnp
from jax import lax
from jax.experimental import pallas as pl
from jax.experimental.pallas import tpu as pltpu
from jax.experimental.pallas import tpu_sc as plsc

_NUM_EMB = 1000000
_D = 32
_R = 4
_SCALE = 10.0
_B = 4096 * 26
_NW = 32
_PER_W = _B // _NW
_C = 128
_NCH = _PER_W // _C


def _sc_body(x_hbm, emb_hbm, u0_hbm, u1_hbm, u2_hbm, u3_hbm, v_hbm, out_hbm,
             idx0, idx1, ixe0, ixe1, im0, im1, e0, e1,
             ua0, ub0, uc0, ud0, ua1, ub1, uc1, ud1, o0, o1, vs,
             sg0, sg1, sw0, sw1):
    wid = lax.axis_index("s") * 2 + lax.axis_index("c")
    base = wid * _PER_W
    u_hbms = (u0_hbm, u1_hbm, u2_hbm, u3_hbm)

    slots = ((idx0, ixe0, im0, e0, (ua0, ub0, uc0, ud0), o0, sg0, sw0),
             (idx1, ixe1, im1, e1, (ua1, ub1, uc1, ud1), o1, sg1, sw1))

    pltpu.sync_copy(v_hbm, vs)
    for r in range(_R):
        for h in range(2):
            vs[r, pl.ds(16 * h, 16)] = vs[r, pl.ds(16 * h, 16)] * _SCALE

    def fire(c, s):
        idx_v, ixe_v, im_v, e_v, u_v, _, sg, _ = slots[s]
        off = pl.multiple_of(base + c * _C, _C)
        pltpu.sync_copy(x_hbm.at[pl.ds(off, _C)], idx_v)
        for q in range(_C // 16):
            iv = idx_v[pl.ds(16 * q, 16)]
            ixe_v[pl.ds(16 * q, 16)] = iv >> 2
            im_v[pl.ds(16 * q, 16)] = iv & 3
        pltpu.async_copy(emb_hbm.at[ixe_v], e_v, sg)
        for r in range(_R):
            pltpu.async_copy(u_hbms[r].at[idx_v], u_v[r], sg)

    def wait_gathers(s):
        idx_v, ixe_v, _, e_v, u_v, _, sg, _ = slots[s]
        pltpu.make_async_copy(emb_hbm.at[ixe_v], e_v, sg).wait()
        for r in range(_R):
            pltpu.make_async_copy(u_hbms[r].at[idx_v], u_v[r], sg).wait()

    def write_out(c, s):
        o_v, sw = slots[s][5], slots[s][7]
        off = pl.multiple_of((base + c * _C) // 4, _C // 4)
        pltpu.async_copy(o_v, out_hbm.at[pl.ds(off, _C // 4)], sw)

    def wait_write(c, s):
        o_v, sw = slots[s][5], slots[s][7]
        off = pl.multiple_of((base + c * _C) // 4, _C // 4)
        pltpu.make_async_copy(
            o_v, out_hbm.at[pl.ds(off, _C // 4)], sw).wait()

    def compute(s):
        _, _, im_v, e_v, u_v, o_v, _, _ = slots[s]

        def row_group(g, _, im_v=im_v, e_v=e_v, u_v=u_v, o_v=o_v):
            ur = [u_v[r][pl.ds(16 * g, 16)] for r in range(_R)]
            imvec = im_v[pl.ds(16 * g, 16)]
            for j in range(16):
                i = g * 16 + j
                mj = imvec[j] * 32
                for h in range(2):
                    acc = e_v[i, pl.ds(mj + 16 * h, 16)] * _SCALE
                    for r in range(_R):
                        acc = acc + ur[r][j] * vs[r, pl.ds(16 * h, 16)]
                    o_v[4 * g + j // 4, pl.ds(32 * (j % 4) + 16 * h, 16)] = acc
            return 0

        lax.fori_loop(0, _C // 16, row_group, 0)

    fire(0, 0)
    fire(1, 1)

    def pair_body(p, _):
        for s in range(2):
            c = p * 2 + s
            wait_gathers(s)

            @pl.when(p >= 1)
            def _():
                wait_write(c - 2, s)

            compute(s)
            write_out(c, s)

            @pl.when(c + 2 < _NCH)
            def _():
                fire(c + 2, s)
        return 0

    lax.fori_loop(0, _NCH // 2, pair_body, 0)
    wait_write(_NCH - 2, 0)
    wait_write(_NCH - 1, 1)


@jax.jit
def kernel(x, emb_weight, lora_U, lora_V):
    x_flat = x.reshape(_B).astype(jnp.int32)
    emb128 = emb_weight.reshape(_NUM_EMB // 4, 128)
    u_t = lora_U.T
    u_cols = [u_t[r] for r in range(_R)]
    v_pad = jnp.zeros((_R, 128), jnp.float32).at[:, :_D].set(lora_V)
    sc_call = pl.kernel(
        _sc_body,
        out_type=jax.ShapeDtypeStruct((_B // 4, 128), jnp.float32),
        mesh=plsc.VectorSubcoreMesh(core_axis_name="c", subcore_axis_name="s"),
        scratch_types=[
            pltpu.VMEM((_C,), jnp.int32),
            pltpu.VMEM((_C,), jnp.int32),
            pltpu.VMEM((_C,), jnp.int32),
            pltpu.VMEM((_C,), jnp.int32),
            pltpu.VMEM((_C,), jnp.int32),
            pltpu.VMEM((_C,), jnp.int32),
            pltpu.VMEM((_C, 128), jnp.float32),
            pltpu.VMEM((_C, 128), jnp.float32),
            pltpu.VMEM((_C,), jnp.float32),
            pltpu.VMEM((_C,), jnp.float32),
            pltpu.VMEM((_C,), jnp.float32),
            pltpu.VMEM((_C,), jnp.float32),
            pltpu.VMEM((_C,), jnp.float32),
            pltpu.VMEM((_C,), jnp.float32),
            pltpu.VMEM((_C,), jnp.float32),
            pltpu.VMEM((_C,), jnp.float32),
            pltpu.VMEM((_C // 4, 128), jnp.float32),
            pltpu.VMEM((_C // 4, 128), jnp.float32),
            pltpu.VMEM((_R, 128), jnp.float32),
            pltpu.SemaphoreType.DMA,
            pltpu.SemaphoreType.DMA,
            pltpu.SemaphoreType.DMA,
            pltpu.SemaphoreType.DMA,
        ],
        compiler_params=pltpu.CompilerParams(use_tc_tiling_on_sc=False,
                                             needs_layout_passes=False),
    )
    out = sc_call(x_flat, emb128, *u_cols, v_pad)
    return out.reshape(x.shape[0], x.shape[1], _D)


# speedup vs baseline: 2.9007x; 1.0227x over previous
import jax
import jax.numpy as jnp
from jax import lax
from jax.experimental import pallas as pl
from jax.experimental.pallas import tpu as pltpu
from jax.experimental.pallas import tpu_sc as plsc

_NUM_EMB = 1000000
_D = 32
_R = 4
_SCALE = 10.0
_B = 4096 * 26
_NW = 32
_PER_W = _B // _NW
_C = 128
_NCH = _PER_W // _C


def _sc_body(x_hbm, emb_hbm, u0_hbm, u1_hbm, u2_hbm, u3_hbm, v_hbm, out_hbm,
             idx0, idx1, e0, e1,
             ua0, ub0, uc0, ud0, ua1, ub1, uc1, ud1, o0, o1, vs,
             sg0, sg1, sw0, sw1):
    wid = lax.axis_index("s") * 2 + lax.axis_index("c")
    base = wid * _PER_W
    u_hbms = (u0_hbm, u1_hbm, u2_hbm, u3_hbm)

    slots = ((idx0, e0, (ua0, ub0, uc0, ud0), o0, sg0, sw0),
             (idx1, e1, (ua1, ub1, uc1, ud1), o1, sg1, sw1))

    pltpu.sync_copy(v_hbm, vs)
    for r in range(_R):
        for h in range(2):
            vs[r, pl.ds(16 * h, 16)] = vs[r, pl.ds(16 * h, 16)] * _SCALE

    def fire(c, s):
        idx_v, e_v, u_v, _, sg, _ = slots[s]
        off = pl.multiple_of(base + c * _C, _C)
        pltpu.sync_copy(x_hbm.at[pl.ds(off, _C)], idx_v)
        pltpu.async_copy(emb_hbm.at[idx_v], e_v, sg)
        for r in range(_R):
            pltpu.async_copy(u_hbms[r].at[idx_v], u_v[r], sg)

    def wait_gathers(s):
        idx_v, e_v, u_v, _, sg, _ = slots[s]
        pltpu.make_async_copy(emb_hbm.at[idx_v], e_v, sg).wait()
        for r in range(_R):
            pltpu.make_async_copy(u_hbms[r].at[idx_v], u_v[r], sg).wait()

    def write_out(c, s):
        o_v, sw = slots[s][3], slots[s][5]
        off = pl.multiple_of((base + c * _C) // 4, _C // 4)
        pltpu.async_copy(o_v, out_hbm.at[pl.ds(off, _C // 4)], sw)

    def wait_write(c, s):
        o_v, sw = slots[s][3], slots[s][5]
        off = pl.multiple_of((base + c * _C) // 4, _C // 4)
        pltpu.make_async_copy(
            o_v, out_hbm.at[pl.ds(off, _C // 4)], sw).wait()

    def compute(s):
        _, e_v, u_v, o_v, _, _ = slots[s]

        def row_group(g, _, e_v=e_v, u_v=u_v, o_v=o_v):
            ur = [u_v[r][pl.ds(16 * g, 16)] for r in range(_R)]
            for j in range(16):
                i = g * 16 + j
                for h in range(2):
                    acc = e_v[i, pl.ds(16 * h, 16)] * _SCALE
                    for r in range(_R):
                        acc = acc + ur[r][j] * vs[r, pl.ds(16 * h, 16)]
                    o_v[4 * g + j // 4, pl.ds(32 * (j % 4) + 16 * h, 16)] = acc
            return 0

        lax.fori_loop(0, _C // 16, row_group, 0)

    fire(0, 0)
    fire(1, 1)

    def pair_body(p, _):
        for s in range(2):
            c = p * 2 + s
            wait_gathers(s)

            @pl.when(p >= 1)
            def _():
                wait_write(c - 2, s)

            compute(s)
            write_out(c, s)

            @pl.when(c + 2 < _NCH)
            def _():
                fire(c + 2, s)
        return 0

    lax.fori_loop(0, _NCH // 2, pair_body, 0)
    wait_write(_NCH - 2, 0)
    wait_write(_NCH - 1, 1)


@jax.jit
def kernel(x, emb_weight, lora_U, lora_V):
    x_flat = x.reshape(_B).astype(jnp.int32)
    emb_pad = jnp.pad(emb_weight, ((0, 0), (0, 128 - _D)))
    u_t = lora_U.T
    u_cols = [u_t[r] for r in range(_R)]
    v_pad = jnp.zeros((_R, 128), jnp.float32).at[:, :_D].set(lora_V)
    sc_call = pl.kernel(
        _sc_body,
        out_type=jax.ShapeDtypeStruct((_B // 4, 128), jnp.float32),
        mesh=plsc.VectorSubcoreMesh(core_axis_name="c", subcore_axis_name="s"),
        scratch_types=[
            pltpu.VMEM((_C,), jnp.int32),
            pltpu.VMEM((_C,), jnp.int32),
            pltpu.VMEM((_C, 128), jnp.float32),
            pltpu.VMEM((_C, 128), jnp.float32),
            pltpu.VMEM((_C,), jnp.float32),
            pltpu.VMEM((_C,), jnp.float32),
            pltpu.VMEM((_C,), jnp.float32),
            pltpu.VMEM((_C,), jnp.float32),
            pltpu.VMEM((_C,), jnp.float32),
            pltpu.VMEM((_C,), jnp.float32),
            pltpu.VMEM((_C,), jnp.float32),
            pltpu.VMEM((_C,), jnp.float32),
            pltpu.VMEM((_C // 4, 128), jnp.float32),
            pltpu.VMEM((_C // 4, 128), jnp.float32),
            pltpu.VMEM((_R, 128), jnp.float32),
            pltpu.SemaphoreType.DMA,
            pltpu.SemaphoreType.DMA,
            pltpu.SemaphoreType.DMA,
            pltpu.SemaphoreType.DMA,
        ],
        compiler_params=pltpu.CompilerParams(use_tc_tiling_on_sc=False,
                                             needs_layout_passes=False),
    )
    out = sc_call(x_flat, emb_pad, *u_cols, v_pad)
    return out.reshape(x.shape[0], x.shape[1], _D)
